# baseline (speedup 1.0000x reference)
import functools

import jax
import jax.numpy as jnp
from jax import lax
from jax.experimental import pallas as pl
from jax.experimental.pallas import tpu as pltpu
from jax.experimental.pallas import tpu_sc as plsc

N = 10000
NPAD = 10240
E = 320000
DF = 128
DE = 16
H = 64
B = 64
HEIGHT = 3

NC = 2
NS = 16
NW = NC * NS
EPW = E // NW
BK = 80
NBLK = EPW // BK
RPS = NPAD // NS

@functools.lru_cache(maxsize=None)
def _mesh():
    return plsc.VectorSubcoreMesh(
        core_axis_name="c", subcore_axis_name="s", num_cores=NC, num_subcores=NS)


def _zero_vmem(buf, rows):
    z = jnp.zeros((16,), jnp.float32)

    @pl.loop(0, rows)
    def _(r):
        for c in range(H // 16):
            buf[r, pl.ds(16 * c, 16)] = z


NB = 5


@functools.lru_cache(maxsize=None)
def _sc_gather_segsum():
    return pl.kernel(
        _sc_gather_segsum_body,
        out_type=jax.ShapeDtypeStruct((NC, NPAD, H), jnp.float32),
        mesh=_mesh(),
        compiler_params=pltpu.CompilerParams(use_tc_tiling_on_sc=False),
        scratch_types=[
            pltpu.VMEM((NBLK, BK), jnp.int32),
            pltpu.VMEM((NBLK, BK), jnp.int32),
            [pltpu.VMEM((BK, H), jnp.float32)] * (2 * NB),
            pltpu.VMEM((BK, H), jnp.float32),
            pltpu.VMEM_SHARED((NPAD, H), jnp.float32),
            [pltpu.SemaphoreType.DMA] * (2 * NB),
            [pltpu.SemaphoreType.DMA] * (2 * NB),
        ],
    )


def _sc_segsum_pipeline(mk_load, didx, acc, rows, gsem, ssem):
    NS2 = 2 * NB
    MAIN = (NBLK // NS2) * NS2

    for b in range(NB):
        mk_load(b, b).start()

    @pl.loop(0, MAIN, step=NS2)
    def _(g):
        for b in range(NS2):
            j = g + b
            mk_load(j, b).wait()
            pltpu.async_copy(rows[b], acc.at[didx.at[j]], ssem[b], add=True)
            nxt = (b + NB) % NS2

            @pl.when(j >= NB)
            def _():
                pltpu.make_async_copy(
                    rows[nxt], acc.at[didx.at[j]], ssem[nxt]).wait()

            mk_load(j + NB, nxt).start()

    for b in range(NBLK - MAIN):
        mk_load(MAIN + b, b).wait()
        pltpu.sync_copy(rows[b], acc.at[didx.at[MAIN + b]], add=True)
    for b in range(NB, NS2):
        pltpu.make_async_copy(rows[b], acc.at[didx.at[0]], ssem[b]).wait()


def _sc_gather_segsum_body(table, srcR, dstR, out, sidx, didx, rows, zbuf, acc,
                           gsem, ssem):
    cid = lax.axis_index("c")
    sid = lax.axis_index("s")
    wid = sid * NC + cid
    _zero_vmem(zbuf, BK)

    @pl.loop(0, RPS // BK)
    def _(b):
        pltpu.sync_copy(zbuf, acc.at[pl.ds(sid * RPS + b * BK, BK)])

    pltpu.sync_copy(srcR.at[wid], sidx)
    pltpu.sync_copy(dstR.at[wid], didx)
    plsc.subcore_barrier()

    def mk_load(j, slot):
        return pltpu.make_async_copy(table.at[sidx.at[j]], rows[slot], gsem[slot])

    _sc_segsum_pipeline(mk_load, didx, acc, rows, gsem, ssem)

    plsc.subcore_barrier()
    pltpu.sync_copy(acc.at[pl.ds(sid * RPS, RPS)],
                    out.at[cid].at[pl.ds(sid * RPS, RPS)])


@functools.lru_cache(maxsize=None)
def _sc_scatter_segsum():
    return pl.kernel(
        _sc_scatter_segsum_body,
        out_type=jax.ShapeDtypeStruct((NC, NPAD, H), jnp.float32),
        mesh=_mesh(),
        compiler_params=pltpu.CompilerParams(use_tc_tiling_on_sc=False),
        scratch_types=[
            pltpu.VMEM((NBLK, BK), jnp.int32),
            [pltpu.VMEM((BK, H), jnp.float32)] * (2 * NB),
            pltpu.VMEM((BK, H), jnp.float32),
            pltpu.VMEM_SHARED((NPAD, H), jnp.float32),
            [pltpu.SemaphoreType.DMA] * (2 * NB),
            [pltpu.SemaphoreType.DMA] * (2 * NB),
        ],
    )


def _sc_scatter_segsum_body(ev, dstR, out, didx, rows, zbuf, acc, gsem, ssem):
    cid = lax.axis_index("c")
    sid = lax.axis_index("s")
    wid = sid * NC + cid
    _zero_vmem(zbuf, BK)

    @pl.loop(0, RPS // BK)
    def _(b):
        pltpu.sync_copy(zbuf, acc.at[pl.ds(sid * RPS + b * BK, BK)])

    pltpu.sync_copy(dstR.at[wid], didx)
    plsc.subcore_barrier()

    def mk_load(j, slot):
        return pltpu.make_async_copy(
            ev.at[pl.ds(wid * EPW + j * BK, BK)], rows[slot], gsem[slot])

    _sc_segsum_pipeline(mk_load, didx, acc, rows, gsem, ssem)

    plsc.subcore_barrier()
    pltpu.sync_copy(acc.at[pl.ds(sid * RPS, RPS)],
                    out.at[cid].at[pl.ds(sid * RPS, RPS)])


@functools.lru_cache(maxsize=None)
def _sc_gather_pair():
    return pl.kernel(
        _sc_gather_pair_body,
        out_type=jax.ShapeDtypeStruct((E, H), jnp.float32),
        mesh=_mesh(),
        compiler_params=pltpu.CompilerParams(use_tc_tiling_on_sc=False),
        scratch_types=[
            pltpu.VMEM((NBLK, BK), jnp.int32),
            pltpu.VMEM((NBLK, BK), jnp.int32),
            [pltpu.VMEM((BK, H), jnp.float32)] * NB,
            [pltpu.VMEM((BK, H), jnp.float32)] * NB,
            [pltpu.VMEM((BK, H), jnp.float32)] * NB,
            [pltpu.SemaphoreType.DMA] * NB,
            [pltpu.SemaphoreType.DMA] * NB,
            [pltpu.SemaphoreType.DMA] * NB,
        ],
    )


def _sc_gather_pair_body(table, srcR, dstR, out, sidx, didx, r1, r2, ob,
                         sem1, sem2, semo):
    cid = lax.axis_index("c")
    sid = lax.axis_index("s")
    wid = sid * NC + cid
    pltpu.sync_copy(srcR.at[wid], sidx)
    pltpu.sync_copy(dstR.at[wid], didx)
    for b in range(NB):
        pltpu.async_copy(table.at[sidx.at[b]], r1[b], sem1[b])
        pltpu.async_copy(table.at[didx.at[b]], r2[b], sem2[b])

    @pl.loop(0, NBLK, step=NB)
    def _(g):
        for b in range(NB):
            j = g + b
            pltpu.make_async_copy(table.at[sidx.at[j]], r1[b], sem1[b]).wait()
            pltpu.make_async_copy(table.at[didx.at[j]], r2[b], sem2[b]).wait()

            @pl.when(j >= NB)
            def _():
                pltpu.make_async_copy(
                    ob[b], out.at[pl.ds(wid * EPW + (j - NB) * BK, BK)],
                    semo[b]).wait()

            @pl.loop(0, BK, unroll=8)
            def _(r):
                for c in range(H // 16):
                    sl = pl.ds(16 * c, 16)
                    ob[b][r, sl] = r1[b][r, sl] + r2[b][r, sl]

            @pl.when(j + NB < NBLK)
            def _():
                pltpu.async_copy(table.at[sidx.at[j + NB]], r1[b], sem1[b])
                pltpu.async_copy(table.at[didx.at[j + NB]], r2[b], sem2[b])

            pltpu.async_copy(
                ob[b], out.at[pl.ds(wid * EPW + j * BK, BK)], semo[b])

    for b in range(NB):
        pltpu.make_async_copy(
            ob[b], out.at[pl.ds(wid * EPW, BK)], semo[b]).wait()


def _ln_relu(y):
    mu = jnp.mean(y, axis=-1, keepdims=True)
    var = jnp.mean((y - mu) ** 2, axis=-1, keepdims=True)
    return jnp.maximum((y - mu) / jnp.sqrt(var + 1e-5), 0.0)


BN = 640
BE = 2000


def _t_init_nodes(x, Wn):
    def body(x_ref, w_ref, o_ref):
        o_ref[...] = _ln_relu(jnp.dot(x_ref[...], w_ref[...],
                                      preferred_element_type=jnp.float32))
    return pl.pallas_call(
        body,
        grid=(NPAD // BN,),
        in_specs=[pl.BlockSpec((BN, DF), lambda i: (i, 0)),
                  pl.BlockSpec((DF, H), lambda i: (0, 0))],
        out_specs=pl.BlockSpec((BN, H), lambda i: (i, 0)),
        out_shape=jax.ShapeDtypeStruct((NPAD, H), jnp.float32),
    )(x, Wn)


def _t_edge_init(ea, We):
    def body(a_ref, w_ref, o_ref):
        o_ref[...] = _ln_relu(jnp.dot(a_ref[...], w_ref[...],
                                      preferred_element_type=jnp.float32))
    return pl.pallas_call(
        body,
        grid=(E // BE,),
        in_specs=[pl.BlockSpec((BE, DE), lambda i: (i, 0)),
                  pl.BlockSpec((DE, H), lambda i: (0, 0))],
        out_specs=pl.BlockSpec((BE, H), lambda i: (i, 0)),
        out_shape=jax.ShapeDtypeStruct((E, H), jnp.float32),
    )(ea, We)


def _t_level(n, P, Q, W, W_next=None):
    two_out = W_next is not None

    def body(*refs):
        if two_out:
            n_ref, p_ref, q_ref, w_ref, wn_ref, o_ref, t_ref = refs
        else:
            n_ref, p_ref, q_ref, w_ref, o_ref = refs
        s = (p_ref[0] + p_ref[1] + q_ref[0] + q_ref[1])
        y = _ln_relu(n_ref[...] + jnp.dot(s, w_ref[...],
                                          preferred_element_type=jnp.float32))
        o_ref[...] = y
        if two_out:
            t_ref[...] = jnp.dot(y, wn_ref[...],
                                 preferred_element_type=jnp.float32)

    in_specs = [pl.BlockSpec((BN, H), lambda i: (i, 0)),
                pl.BlockSpec((NC, BN, H), lambda i: (0, i, 0)),
                pl.BlockSpec((NC, BN, H), lambda i: (0, i, 0)),
                pl.BlockSpec((H, H), lambda i: (0, 0))]
    out_spec = pl.BlockSpec((BN, H), lambda i: (i, 0))
    shp = jax.ShapeDtypeStruct((NPAD, H), jnp.float32)
    if two_out:
        return pl.pallas_call(
            body, grid=(NPAD // BN,),
            in_specs=in_specs + [pl.BlockSpec((H, H), lambda i: (0, 0))],
            out_specs=(out_spec, out_spec), out_shape=(shp, shp),
        )(n, P, Q, W, W_next)
    return pl.pallas_call(
        body, grid=(NPAD // BN,),
        in_specs=in_specs, out_specs=out_spec, out_shape=shp,
    )(n, P, Q, W)


def _t_segnode(top, nbR):
    def body(t_ref, nb_ref, o_ref):
        i = pl.program_id(0)
        oh = (lax.broadcasted_iota(jnp.int32, (B, BN), 0)
              == nb_ref[0, 0, :][None, :]).astype(jnp.float32)
        part = jnp.dot(oh, t_ref[...], preferred_element_type=jnp.float32)

        @pl.when(i == 0)
        def _():
            o_ref[...] = jnp.zeros_like(o_ref)

        o_ref[...] += part

    return pl.pallas_call(
        body, grid=(NPAD // BN,),
        in_specs=[pl.BlockSpec((BN, H), lambda i: (i, 0)),
                  pl.BlockSpec((1, 1, BN), lambda i: (i, 0, 0))],
        out_specs=pl.BlockSpec((B, H), lambda i: (0, 0)),
        out_shape=jax.ShapeDtypeStruct((B, H), jnp.float32),
    )(top, nbR)


def _t_assign(top, nbR, node_agg, W_as):
    def body(t_ref, nb_ref, na_ref, w_ref, o_ref):
        M = jnp.dot(na_ref[...], w_ref[...], preferred_element_type=jnp.float32)
        oh = (nb_ref[0, 0, :][:, None]
              == lax.broadcasted_iota(jnp.int32, (BN, B), 1)).astype(jnp.float32)
        ctx = jnp.dot(oh, M, preferred_element_type=jnp.float32)
        o_ref[...] = _ln_relu(t_ref[...] + ctx)

    return pl.pallas_call(
        body, grid=(NPAD // BN,),
        in_specs=[pl.BlockSpec((BN, H), lambda i: (i, 0)),
                  pl.BlockSpec((1, 1, BN), lambda i: (i, 0, 0)),
                  pl.BlockSpec((B, H), lambda i: (0, 0)),
                  pl.BlockSpec((H, H), lambda i: (0, 0))],
        out_specs=pl.BlockSpec((BN, H), lambda i: (i, 0)),
        out_shape=jax.ShapeDtypeStruct((NPAD, H), jnp.float32),
    )(top, nbR, node_agg, W_as)


def _t_edge_update(e_prev, g, W):
    def body(e_ref, g_ref, w_ref, o_ref):
        o_ref[...] = _ln_relu(jnp.dot(e_ref[...], w_ref[...],
                                      preferred_element_type=jnp.float32)
                              + g_ref[...])
    return pl.pallas_call(
        body, grid=(E // BE,),
        in_specs=[pl.BlockSpec((BE, H), lambda i: (i, 0)),
                  pl.BlockSpec((BE, H), lambda i: (i, 0)),
                  pl.BlockSpec((H, H), lambda i: (0, 0))],
        out_specs=pl.BlockSpec((BE, H), lambda i: (i, 0)),
        out_shape=jax.ShapeDtypeStruct((E, H), jnp.float32),
    )(e_prev, g, W)


def _t_edge_final(e_prev, g, W, ebR):
    def body(e_ref, g_ref, w_ref, eb_ref, o_ref):
        i = pl.program_id(0)
        e2 = _ln_relu(jnp.dot(e_ref[...], w_ref[...],
                              preferred_element_type=jnp.float32) + g_ref[...])
        oh = (lax.broadcasted_iota(jnp.int32, (B, BE), 0)
              == eb_ref[0, 0, :][None, :]).astype(jnp.float32)
        part = jnp.dot(oh, e2, preferred_element_type=jnp.float32)

        @pl.when(i == 0)
        def _():
            o_ref[...] = jnp.zeros_like(o_ref)

        o_ref[...] += part

    return pl.pallas_call(
        body, grid=(E // BE,),
        in_specs=[pl.BlockSpec((BE, H), lambda i: (i, 0)),
                  pl.BlockSpec((BE, H), lambda i: (i, 0)),
                  pl.BlockSpec((H, H), lambda i: (0, 0)),
                  pl.BlockSpec((1, 1, BE), lambda i: (i, 0, 0))],
        out_specs=pl.BlockSpec((B, H), lambda i: (0, 0)),
        out_shape=jax.ShapeDtypeStruct((B, H), jnp.float32),
    )(e_prev, g, W, ebR)


def _t_final(na, ea, W1, W2, b1, W_out, b_out):
    def body(na_ref, ea_ref, w1_ref, w2_ref, b1_ref, wo_ref, bo_ref, o_ref):
        h = _ln_relu(jnp.dot(na_ref[...], w1_ref[...],
                             preferred_element_type=jnp.float32)
                     + jnp.dot(ea_ref[...], w2_ref[...],
                               preferred_element_type=jnp.float32)
                     + b1_ref[...])
        o_ref[...] = jnp.dot(h, wo_ref[...],
                             preferred_element_type=jnp.float32) + bo_ref[...]

    spec = pl.BlockSpec((B, H), lambda: (0, 0))
    wspec = pl.BlockSpec((H, H), lambda: (0, 0))
    return pl.pallas_call(
        body,
        in_specs=[spec, spec, wspec, wspec,
                  pl.BlockSpec((1, H), lambda: (0, 0)),
                  pl.BlockSpec((H, H), lambda: (0, 0)),
                  pl.BlockSpec((1, H), lambda: (0, 0))],
        out_specs=pl.BlockSpec((B, H), lambda: (0, 0)),
        out_shape=jax.ShapeDtypeStruct((B, H), jnp.float32),
    )(na, ea, W1, W2, b1, W_out, b_out)


def kernel(x, edge_attr, Wn, We, W_bu, W_eu, W_assign, W1, W2, b1, W_out, b_out,
           edge_index, node_batch, edge_batch):
    src = edge_index[0].astype(jnp.int32)
    dst = edge_index[1].astype(jnp.int32)
    srcR = src.reshape(NW, NBLK, BK)
    dstR = dst.reshape(NW, NBLK, BK)
    xp = jnp.pad(x, ((0, NPAD - N), (0, 0)))
    nbR = jnp.pad(node_batch.astype(jnp.int32), (0, NPAD - N),
                  constant_values=B).reshape(NPAD // BN, 1, BN)
    ebR = edge_batch.astype(jnp.int32).reshape(E // BE, 1, BE)
    W1p = jnp.pad(W_out, ((0, 0), (0, H - 1)))
    b1p = b1.reshape(1, H)
    bop = jnp.pad(b_out, (0, H - 1)).reshape(1, H)

    n = _t_init_nodes(xp, Wn)
    e0 = _t_edge_init(edge_attr, We)
    Q = _sc_scatter_segsum()(e0, dstR)

    for lvl in range(HEIGHT):
        P = _sc_gather_segsum()(n, srcR, dstR)
        if lvl == HEIGHT - 1:
            n, t0 = _t_level(n, P, Q, W_bu[0, lvl], W_eu[0])
        else:
            n = _t_level(n, P, Q, W_bu[0, lvl])
    top0 = n

    g0 = _sc_gather_pair()(t0, srcR, dstR)
    e1 = _t_edge_update(e0, g0, W_eu[0])
    Q1 = _sc_scatter_segsum()(e1, dstR)

    na0 = _t_segnode(top0, nbR)
    n = _t_assign(top0, nbR, na0, W_assign[0])

    for lvl in range(HEIGHT):
        P = _sc_gather_segsum()(n, srcR, dstR)
        if lvl == HEIGHT - 1:
            n, t1 = _t_level(n, P, Q1, W_bu[1, lvl], W_eu[1])
        else:
            n = _t_level(n, P, Q1, W_bu[1, lvl])
    top1 = n

    g1 = _sc_gather_pair()(t1, srcR, dstR)
    edge_agg = _t_edge_final(e1, g1, W_eu[1], ebR)
    naf = _t_segnode(top1, nbR)

    out = _t_final(naf, edge_agg, W1, W2, b1p, W1p, bop)
    return out[:, :1]


# speedup vs baseline: 1.0139x; 1.0139x over previous
import functools

import jax
import jax.numpy as jnp
from jax import lax
from jax.experimental import pallas as pl
from jax.experimental.pallas import tpu as pltpu
from jax.experimental.pallas import tpu_sc as plsc

N = 10000
NPAD = 10240
E = 320000
DF = 128
DE = 16
H = 64
B = 64
HEIGHT = 3

NC = 2
NS = 16
NW = NC * NS
EPW = E // NW
BK = 80
NBLK = EPW // BK
RPS = NPAD // NS

@functools.lru_cache(maxsize=None)
def _mesh():
    return plsc.VectorSubcoreMesh(
        core_axis_name="c", subcore_axis_name="s", num_cores=NC, num_subcores=NS)


def _zero_vmem(buf, rows):
    z = jnp.zeros((16,), jnp.float32)

    @pl.loop(0, rows)
    def _(r):
        for c in range(H // 16):
            buf[r, pl.ds(16 * c, 16)] = z


NB = 5


@functools.lru_cache(maxsize=None)
def _sc_gather_segsum():
    return pl.kernel(
        _sc_gather_segsum_body,
        out_type=jax.ShapeDtypeStruct((NC, NPAD, H), jnp.float32),
        mesh=_mesh(),
        compiler_params=pltpu.CompilerParams(use_tc_tiling_on_sc=False),
        scratch_types=[
            pltpu.VMEM((NBLK, BK), jnp.int32),
            pltpu.VMEM((NBLK, BK), jnp.int32),
            [pltpu.VMEM((BK, H), jnp.float32)] * NB,
            pltpu.VMEM((BK, H), jnp.float32),
            pltpu.VMEM_SHARED((NPAD, H), jnp.float32),
            [pltpu.SemaphoreType.DMA] * NB,
        ],
    )


def _sc_gather_segsum_body(table, srcR, dstR, out, sidx, didx, rows, zbuf, acc,
                           sems):
    cid = lax.axis_index("c")
    sid = lax.axis_index("s")
    wid = sid * NC + cid
    _zero_vmem(zbuf, BK)

    @pl.loop(0, RPS // BK)
    def _(b):
        pltpu.sync_copy(zbuf, acc.at[pl.ds(sid * RPS + b * BK, BK)])

    pltpu.sync_copy(srcR.at[wid], sidx)
    pltpu.sync_copy(dstR.at[wid], didx)
    for b in range(NB):
        pltpu.async_copy(table.at[sidx.at[b]], rows[b], sems[b])
    plsc.subcore_barrier()

    @pl.loop(0, NBLK, step=NB)
    def _(g):
        for b in range(NB):
            j = g + b
            pltpu.make_async_copy(table.at[sidx.at[j]], rows[b], sems[b]).wait()
            pltpu.sync_copy(rows[b], acc.at[didx.at[j]], add=True)

            @pl.when(j + NB < NBLK)
            def _():
                pltpu.async_copy(table.at[sidx.at[j + NB]], rows[b], sems[b])

    plsc.subcore_barrier()
    pltpu.sync_copy(acc.at[pl.ds(sid * RPS, RPS)],
                    out.at[cid].at[pl.ds(sid * RPS, RPS)])


@functools.lru_cache(maxsize=None)
def _sc_scatter_segsum():
    return pl.kernel(
        _sc_scatter_segsum_body,
        out_type=jax.ShapeDtypeStruct((NC, NPAD, H), jnp.float32),
        mesh=_mesh(),
        compiler_params=pltpu.CompilerParams(use_tc_tiling_on_sc=False),
        scratch_types=[
            pltpu.VMEM((NBLK, BK), jnp.int32),
            [pltpu.VMEM((BK, H), jnp.float32)] * NB,
            pltpu.VMEM((BK, H), jnp.float32),
            pltpu.VMEM_SHARED((NPAD, H), jnp.float32),
            [pltpu.SemaphoreType.DMA] * NB,
        ],
    )


def _sc_scatter_segsum_body(ev, dstR, out, didx, rows, zbuf, acc, sems):
    cid = lax.axis_index("c")
    sid = lax.axis_index("s")
    wid = sid * NC + cid
    _zero_vmem(zbuf, BK)

    @pl.loop(0, RPS // BK)
    def _(b):
        pltpu.sync_copy(zbuf, acc.at[pl.ds(sid * RPS + b * BK, BK)])

    pltpu.sync_copy(dstR.at[wid], didx)
    for b in range(NB):
        pltpu.async_copy(ev.at[pl.ds(wid * EPW + b * BK, BK)], rows[b], sems[b])
    plsc.subcore_barrier()

    @pl.loop(0, NBLK, step=NB)
    def _(g):
        for b in range(NB):
            j = g + b
            pltpu.make_async_copy(
                ev.at[pl.ds(wid * EPW + j * BK, BK)], rows[b], sems[b]).wait()
            pltpu.sync_copy(rows[b], acc.at[didx.at[j]], add=True)

            @pl.when(j + NB < NBLK)
            def _():
                pltpu.async_copy(
                    ev.at[pl.ds(wid * EPW + (j + NB) * BK, BK)], rows[b], sems[b])

    plsc.subcore_barrier()
    pltpu.sync_copy(acc.at[pl.ds(sid * RPS, RPS)],
                    out.at[cid].at[pl.ds(sid * RPS, RPS)])


@functools.lru_cache(maxsize=None)
def _sc_gather_e_segsum():
    return pl.kernel(
        _sc_gather_e_segsum_body,
        out_type=(jax.ShapeDtypeStruct((NC, NPAD, H), jnp.float32),
                  jax.ShapeDtypeStruct((NC, NPAD, H), jnp.float32)),
        mesh=_mesh(),
        compiler_params=pltpu.CompilerParams(use_tc_tiling_on_sc=False),
        scratch_types=[
            pltpu.VMEM((NBLK, BK), jnp.int32),
            pltpu.VMEM((NBLK, BK), jnp.int32),
            [pltpu.VMEM((BK, H), jnp.float32)] * NB,
            [pltpu.VMEM((BK, H), jnp.float32)] * NB,
            pltpu.VMEM((BK, H), jnp.float32),
            pltpu.VMEM_SHARED((NPAD, H), jnp.float32),
            pltpu.VMEM_SHARED((NPAD, H), jnp.float32),
            [pltpu.SemaphoreType.DMA] * NB,
            [pltpu.SemaphoreType.DMA] * NB,
        ],
    )


def _sc_gather_e_segsum_body(table, ev, srcR, dstR, outP, outQ, sidx, didx,
                             rows, erow, zbuf, accP, accQ, gsem, esem):
    cid = lax.axis_index("c")
    sid = lax.axis_index("s")
    wid = sid * NC + cid
    _zero_vmem(zbuf, BK)

    @pl.loop(0, RPS // BK)
    def _(b):
        pltpu.sync_copy(zbuf, accP.at[pl.ds(sid * RPS + b * BK, BK)])
        pltpu.sync_copy(zbuf, accQ.at[pl.ds(sid * RPS + b * BK, BK)])

    pltpu.sync_copy(srcR.at[wid], sidx)
    pltpu.sync_copy(dstR.at[wid], didx)
    for b in range(NB):
        pltpu.async_copy(table.at[sidx.at[b]], rows[b], gsem[b])
        pltpu.async_copy(ev.at[pl.ds(wid * EPW + b * BK, BK)], erow[b], esem[b])
    plsc.subcore_barrier()

    @pl.loop(0, NBLK, step=NB)
    def _(g):
        for b in range(NB):
            j = g + b
            pltpu.make_async_copy(table.at[sidx.at[j]], rows[b], gsem[b]).wait()
            pltpu.sync_copy(rows[b], accP.at[didx.at[j]], add=True)
            pltpu.make_async_copy(
                ev.at[pl.ds(wid * EPW + j * BK, BK)], erow[b], esem[b]).wait()
            pltpu.sync_copy(erow[b], accQ.at[didx.at[j]], add=True)

            @pl.when(j + NB < NBLK)
            def _():
                pltpu.async_copy(table.at[sidx.at[j + NB]], rows[b], gsem[b])
                pltpu.async_copy(
                    ev.at[pl.ds(wid * EPW + (j + NB) * BK, BK)], erow[b], esem[b])

    plsc.subcore_barrier()
    pltpu.sync_copy(accP.at[pl.ds(sid * RPS, RPS)],
                    outP.at[cid].at[pl.ds(sid * RPS, RPS)])
    pltpu.sync_copy(accQ.at[pl.ds(sid * RPS, RPS)],
                    outQ.at[cid].at[pl.ds(sid * RPS, RPS)])


@functools.lru_cache(maxsize=None)
def _sc_gather_pair():
    return pl.kernel(
        _sc_gather_pair_body,
        out_type=jax.ShapeDtypeStruct((E, H), jnp.float32),
        mesh=_mesh(),
        compiler_params=pltpu.CompilerParams(use_tc_tiling_on_sc=False),
        scratch_types=[
            pltpu.VMEM((NBLK, BK), jnp.int32),
            pltpu.VMEM((NBLK, BK), jnp.int32),
            [pltpu.VMEM((BK, H), jnp.float32)] * NB,
            [pltpu.VMEM((BK, H), jnp.float32)] * NB,
            [pltpu.VMEM((BK, H), jnp.float32)] * NB,
            [pltpu.SemaphoreType.DMA] * NB,
            [pltpu.SemaphoreType.DMA] * NB,
            [pltpu.SemaphoreType.DMA] * NB,
        ],
    )


def _sc_gather_pair_body(table, srcR, dstR, out, sidx, didx, r1, r2, ob,
                         sem1, sem2, semo):
    cid = lax.axis_index("c")
    sid = lax.axis_index("s")
    wid = sid * NC + cid
    pltpu.sync_copy(srcR.at[wid], sidx)
    pltpu.sync_copy(dstR.at[wid], didx)
    for b in range(NB):
        pltpu.async_copy(table.at[sidx.at[b]], r1[b], sem1[b])
        pltpu.async_copy(table.at[didx.at[b]], r2[b], sem2[b])

    @pl.loop(0, NBLK, step=NB)
    def _(g):
        for b in range(NB):
            j = g + b
            pltpu.make_async_copy(table.at[sidx.at[j]], r1[b], sem1[b]).wait()
            pltpu.make_async_copy(table.at[didx.at[j]], r2[b], sem2[b]).wait()

            @pl.when(j >= NB)
            def _():
                pltpu.make_async_copy(
                    ob[b], out.at[pl.ds(wid * EPW + (j - NB) * BK, BK)],
                    semo[b]).wait()

            @pl.loop(0, BK, unroll=8)
            def _(r):
                for c in range(H // 16):
                    sl = pl.ds(16 * c, 16)
                    ob[b][r, sl] = r1[b][r, sl] + r2[b][r, sl]

            @pl.when(j + NB < NBLK)
            def _():
                pltpu.async_copy(table.at[sidx.at[j + NB]], r1[b], sem1[b])
                pltpu.async_copy(table.at[didx.at[j + NB]], r2[b], sem2[b])

            pltpu.async_copy(
                ob[b], out.at[pl.ds(wid * EPW + j * BK, BK)], semo[b])

    for b in range(NB):
        pltpu.make_async_copy(
            ob[b], out.at[pl.ds(wid * EPW, BK)], semo[b]).wait()


def _ln_relu(y):
    mu = jnp.mean(y, axis=-1, keepdims=True)
    var = jnp.mean((y - mu) ** 2, axis=-1, keepdims=True)
    return jnp.maximum((y - mu) / jnp.sqrt(var + 1e-5), 0.0)


BN = 640
BE = 2000


def _t_init_nodes(x, Wn):
    def body(x_ref, w_ref, o_ref):
        o_ref[...] = _ln_relu(jnp.dot(x_ref[...], w_ref[...],
                                      preferred_element_type=jnp.float32))
    return pl.pallas_call(
        body,
        grid=(NPAD // BN,),
        in_specs=[pl.BlockSpec((BN, DF), lambda i: (i, 0)),
                  pl.BlockSpec((DF, H), lambda i: (0, 0))],
        out_specs=pl.BlockSpec((BN, H), lambda i: (i, 0)),
        out_shape=jax.ShapeDtypeStruct((NPAD, H), jnp.float32),
    )(x, Wn)


def _t_edge_init(ea, We):
    def body(a_ref, w_ref, o_ref):
        o_ref[...] = _ln_relu(jnp.dot(a_ref[...], w_ref[...],
                                      preferred_element_type=jnp.float32))
    return pl.pallas_call(
        body,
        grid=(E // BE,),
        in_specs=[pl.BlockSpec((BE, DE), lambda i: (i, 0)),
                  pl.BlockSpec((DE, H), lambda i: (0, 0))],
        out_specs=pl.BlockSpec((BE, H), lambda i: (i, 0)),
        out_shape=jax.ShapeDtypeStruct((E, H), jnp.float32),
    )(ea, We)


def _t_level(n, P, Q, W, W_next=None):
    two_out = W_next is not None

    def body(*refs):
        if two_out:
            n_ref, p_ref, q_ref, w_ref, wn_ref, o_ref, t_ref = refs
        else:
            n_ref, p_ref, q_ref, w_ref, o_ref = refs
        s = (p_ref[0] + p_ref[1] + q_ref[0] + q_ref[1])
        y = _ln_relu(n_ref[...] + jnp.dot(s, w_ref[...],
                                          preferred_element_type=jnp.float32))
        o_ref[...] = y
        if two_out:
            t_ref[...] = jnp.dot(y, wn_ref[...],
                                 preferred_element_type=jnp.float32)

    in_specs = [pl.BlockSpec((BN, H), lambda i: (i, 0)),
                pl.BlockSpec((NC, BN, H), lambda i: (0, i, 0)),
                pl.BlockSpec((NC, BN, H), lambda i: (0, i, 0)),
                pl.BlockSpec((H, H), lambda i: (0, 0))]
    out_spec = pl.BlockSpec((BN, H), lambda i: (i, 0))
    shp = jax.ShapeDtypeStruct((NPAD, H), jnp.float32)
    if two_out:
        return pl.pallas_call(
            body, grid=(NPAD // BN,),
            in_specs=in_specs + [pl.BlockSpec((H, H), lambda i: (0, 0))],
            out_specs=(out_spec, out_spec), out_shape=(shp, shp),
        )(n, P, Q, W, W_next)
    return pl.pallas_call(
        body, grid=(NPAD // BN,),
        in_specs=in_specs, out_specs=out_spec, out_shape=shp,
    )(n, P, Q, W)


def _t_segnode(top, nbR):
    def body(t_ref, nb_ref, o_ref):
        i = pl.program_id(0)
        oh = (lax.broadcasted_iota(jnp.int32, (B, BN), 0)
              == nb_ref[0, 0, :][None, :]).astype(jnp.float32)
        part = jnp.dot(oh, t_ref[...], preferred_element_type=jnp.float32)

        @pl.when(i == 0)
        def _():
            o_ref[...] = jnp.zeros_like(o_ref)

        o_ref[...] += part

    return pl.pallas_call(
        body, grid=(NPAD // BN,),
        in_specs=[pl.BlockSpec((BN, H), lambda i: (i, 0)),
                  pl.BlockSpec((1, 1, BN), lambda i: (i, 0, 0))],
        out_specs=pl.BlockSpec((B, H), lambda i: (0, 0)),
        out_shape=jax.ShapeDtypeStruct((B, H), jnp.float32),
    )(top, nbR)


def _t_assign(top, nbR, node_agg, W_as):
    def body(t_ref, nb_ref, na_ref, w_ref, o_ref):
        M = jnp.dot(na_ref[...], w_ref[...], preferred_element_type=jnp.float32)
        oh = (nb_ref[0, 0, :][:, None]
              == lax.broadcasted_iota(jnp.int32, (BN, B), 1)).astype(jnp.float32)
        ctx = jnp.dot(oh, M, preferred_element_type=jnp.float32)
        o_ref[...] = _ln_relu(t_ref[...] + ctx)

    return pl.pallas_call(
        body, grid=(NPAD // BN,),
        in_specs=[pl.BlockSpec((BN, H), lambda i: (i, 0)),
                  pl.BlockSpec((1, 1, BN), lambda i: (i, 0, 0)),
                  pl.BlockSpec((B, H), lambda i: (0, 0)),
                  pl.BlockSpec((H, H), lambda i: (0, 0))],
        out_specs=pl.BlockSpec((BN, H), lambda i: (i, 0)),
        out_shape=jax.ShapeDtypeStruct((NPAD, H), jnp.float32),
    )(top, nbR, node_agg, W_as)


def _t_edge_update(e_prev, g, W):
    def body(e_ref, g_ref, w_ref, o_ref):
        o_ref[...] = _ln_relu(jnp.dot(e_ref[...], w_ref[...],
                                      preferred_element_type=jnp.float32)
                              + g_ref[...])
    return pl.pallas_call(
        body, grid=(E // BE,),
        in_specs=[pl.BlockSpec((BE, H), lambda i: (i, 0)),
                  pl.BlockSpec((BE, H), lambda i: (i, 0)),
                  pl.BlockSpec((H, H), lambda i: (0, 0))],
        out_specs=pl.BlockSpec((BE, H), lambda i: (i, 0)),
        out_shape=jax.ShapeDtypeStruct((E, H), jnp.float32),
    )(e_prev, g, W)


def _t_edge_final(e_prev, g, W, ebR):
    def body(e_ref, g_ref, w_ref, eb_ref, o_ref):
        i = pl.program_id(0)
        e2 = _ln_relu(jnp.dot(e_ref[...], w_ref[...],
                              preferred_element_type=jnp.float32) + g_ref[...])
        oh = (lax.broadcasted_iota(jnp.int32, (B, BE), 0)
              == eb_ref[0, 0, :][None, :]).astype(jnp.float32)
        part = jnp.dot(oh, e2, preferred_element_type=jnp.float32)

        @pl.when(i == 0)
        def _():
            o_ref[...] = jnp.zeros_like(o_ref)

        o_ref[...] += part

    return pl.pallas_call(
        body, grid=(E // BE,),
        in_specs=[pl.BlockSpec((BE, H), lambda i: (i, 0)),
                  pl.BlockSpec((BE, H), lambda i: (i, 0)),
                  pl.BlockSpec((H, H), lambda i: (0, 0)),
                  pl.BlockSpec((1, 1, BE), lambda i: (i, 0, 0))],
        out_specs=pl.BlockSpec((B, H), lambda i: (0, 0)),
        out_shape=jax.ShapeDtypeStruct((B, H), jnp.float32),
    )(e_prev, g, W, ebR)


def _t_final(na, ea, W1, W2, b1, W_out, b_out):
    def body(na_ref, ea_ref, w1_ref, w2_ref, b1_ref, wo_ref, bo_ref, o_ref):
        h = _ln_relu(jnp.dot(na_ref[...], w1_ref[...],
                             preferred_element_type=jnp.float32)
                     + jnp.dot(ea_ref[...], w2_ref[...],
                               preferred_element_type=jnp.float32)
                     + b1_ref[...])
        o_ref[...] = jnp.dot(h, wo_ref[...],
                             preferred_element_type=jnp.float32) + bo_ref[...]

    spec = pl.BlockSpec((B, H), lambda: (0, 0))
    wspec = pl.BlockSpec((H, H), lambda: (0, 0))
    return pl.pallas_call(
        body,
        in_specs=[spec, spec, wspec, wspec,
                  pl.BlockSpec((1, H), lambda: (0, 0)),
                  pl.BlockSpec((H, H), lambda: (0, 0)),
                  pl.BlockSpec((1, H), lambda: (0, 0))],
        out_specs=pl.BlockSpec((B, H), lambda: (0, 0)),
        out_shape=jax.ShapeDtypeStruct((B, H), jnp.float32),
    )(na, ea, W1, W2, b1, W_out, b_out)


def kernel(x, edge_attr, Wn, We, W_bu, W_eu, W_assign, W1, W2, b1, W_out, b_out,
           edge_index, node_batch, edge_batch):
    src = edge_index[0].astype(jnp.int32)
    dst = edge_index[1].astype(jnp.int32)
    srcR = src.reshape(NW, NBLK, BK)
    dstR = dst.reshape(NW, NBLK, BK)
    xp = jnp.pad(x, ((0, NPAD - N), (0, 0)))
    nbR = jnp.pad(node_batch.astype(jnp.int32), (0, NPAD - N),
                  constant_values=B).reshape(NPAD // BN, 1, BN)
    ebR = edge_batch.astype(jnp.int32).reshape(E // BE, 1, BE)
    W1p = jnp.pad(W_out, ((0, 0), (0, H - 1)))
    b1p = b1.reshape(1, H)
    bop = jnp.pad(b_out, (0, H - 1)).reshape(1, H)

    n = _t_init_nodes(xp, Wn)
    e0 = _t_edge_init(edge_attr, We)

    Q = _sc_scatter_segsum()(e0, dstR)

    for lvl in range(HEIGHT):
        P = _sc_gather_segsum()(n, srcR, dstR)
        if lvl == HEIGHT - 1:
            n, t0 = _t_level(n, P, Q, W_bu[0, lvl], W_eu[0])
        else:
            n = _t_level(n, P, Q, W_bu[0, lvl])
    top0 = n

    g0 = _sc_gather_pair()(t0, srcR, dstR)
    e1 = _t_edge_update(e0, g0, W_eu[0])

    na0 = _t_segnode(top0, nbR)
    n = _t_assign(top0, nbR, na0, W_assign[0])

    Q1 = _sc_scatter_segsum()(e1, dstR)

    for lvl in range(HEIGHT):
        P = _sc_gather_segsum()(n, srcR, dstR)
        if lvl == HEIGHT - 1:
            n, t1 = _t_level(n, P, Q1, W_bu[1, lvl], W_eu[1])
        else:
            n = _t_level(n, P, Q1, W_bu[1, lvl])
    top1 = n

    g1 = _sc_gather_pair()(t1, srcR, dstR)
    edge_agg = _t_edge_final(e1, g1, W_eu[1], ebR)
    naf = _t_segnode(top1, nbR)

    out = _t_final(naf, edge_agg, W1, W2, b1p, W1p, bop)
    return out[:, :1]


# speedup vs baseline: 1.4517x; 1.4318x over previous
import functools

import jax
import jax.numpy as jnp
from jax import lax
from jax.experimental import pallas as pl
from jax.experimental.pallas import tpu as pltpu
from jax.experimental.pallas import tpu_sc as plsc

N = 10000
NPAD = 10240
E = 320000
E2 = E // 2
DF = 128
DE = 16
H = 64
B = 64
HEIGHT = 3

NC = 2
NS = 16
NW = NC * NS
EPW = E // NW
BK = 80
NBLK = EPW // BK
RPS = NPAD // NS

@functools.lru_cache(maxsize=None)
def _mesh():
    return plsc.VectorSubcoreMesh(
        core_axis_name="c", subcore_axis_name="s", num_cores=NC, num_subcores=NS)


def _zero_vmem(buf, rows):
    z = jnp.zeros((16,), jnp.float32)

    @pl.loop(0, rows)
    def _(r):
        for c in range(H // 16):
            buf[r, pl.ds(16 * c, 16)] = z


NB = 5


@functools.lru_cache(maxsize=None)
def _sc_gather_segsum():
    return pl.kernel(
        _sc_gather_segsum_body,
        out_type=jax.ShapeDtypeStruct((NC, NPAD, H), jnp.float32),
        mesh=_mesh(),
        compiler_params=pltpu.CompilerParams(use_tc_tiling_on_sc=False),
        scratch_types=[
            pltpu.VMEM((NBLK, BK), jnp.int32),
            pltpu.VMEM((NBLK, BK), jnp.int32),
            [pltpu.VMEM((BK, H), jnp.float32)] * NB,
            pltpu.VMEM((BK, H), jnp.float32),
            pltpu.VMEM_SHARED((NPAD, H), jnp.float32),
            [pltpu.SemaphoreType.DMA] * NB,
        ],
    )


def _sc_gather_segsum_body(table, srcR, dstR, out, sidx, didx, rows, zbuf, acc,
                           sems):
    cid = lax.axis_index("c")
    sid = lax.axis_index("s")
    wid = sid * NC + cid
    _zero_vmem(zbuf, BK)

    @pl.loop(0, RPS // BK)
    def _(b):
        pltpu.sync_copy(zbuf, acc.at[pl.ds(sid * RPS + b * BK, BK)])

    pltpu.sync_copy(srcR.at[wid], sidx)
    pltpu.sync_copy(dstR.at[wid], didx)
    for b in range(NB):
        pltpu.async_copy(table.at[sidx.at[b]], rows[b], sems[b])
    plsc.subcore_barrier()

    @pl.loop(0, NBLK, step=NB)
    def _(g):
        for b in range(NB):
            j = g + b
            pltpu.make_async_copy(table.at[sidx.at[j]], rows[b], sems[b]).wait()
            pltpu.sync_copy(rows[b], acc.at[didx.at[j]], add=True)

            @pl.when(j + NB < NBLK)
            def _():
                pltpu.async_copy(table.at[sidx.at[j + NB]], rows[b], sems[b])

    plsc.subcore_barrier()
    pltpu.sync_copy(acc.at[pl.ds(sid * RPS, RPS)],
                    out.at[cid].at[pl.ds(sid * RPS, RPS)])


@functools.lru_cache(maxsize=None)
def _sc_scatter_segsum():
    return pl.kernel(
        _sc_scatter_segsum_body,
        out_type=jax.ShapeDtypeStruct((NC, NPAD, H), jnp.float32),
        mesh=_mesh(),
        compiler_params=pltpu.CompilerParams(use_tc_tiling_on_sc=False),
        scratch_types=[
            pltpu.VMEM((NBLK, BK), jnp.int32),
            [pltpu.VMEM((BK, H), jnp.float32)] * NB,
            pltpu.VMEM((BK, H), jnp.float32),
            pltpu.VMEM_SHARED((NPAD, H), jnp.float32),
            [pltpu.SemaphoreType.DMA] * NB,
        ],
    )


def _sc_scatter_segsum_body(ev, dstR, out, didx, rows, zbuf, acc, sems):
    cid = lax.axis_index("c")
    sid = lax.axis_index("s")
    wid = sid * NC + cid
    _zero_vmem(zbuf, BK)

    @pl.loop(0, RPS // BK)
    def _(b):
        pltpu.sync_copy(zbuf, acc.at[pl.ds(sid * RPS + b * BK, BK)])

    pltpu.sync_copy(dstR.at[wid], didx)
    for b in range(NB):
        pltpu.async_copy(ev.at[pl.ds(wid * EPW + b * BK, BK)], rows[b], sems[b])
    plsc.subcore_barrier()

    @pl.loop(0, NBLK, step=NB)
    def _(g):
        for b in range(NB):
            j = g + b
            pltpu.make_async_copy(
                ev.at[pl.ds(wid * EPW + j * BK, BK)], rows[b], sems[b]).wait()
            pltpu.sync_copy(rows[b], acc.at[didx.at[j]], add=True)

            @pl.when(j + NB < NBLK)
            def _():
                pltpu.async_copy(
                    ev.at[pl.ds(wid * EPW + (j + NB) * BK, BK)], rows[b], sems[b])

    plsc.subcore_barrier()
    pltpu.sync_copy(acc.at[pl.ds(sid * RPS, RPS)],
                    out.at[cid].at[pl.ds(sid * RPS, RPS)])


@functools.lru_cache(maxsize=None)
def _sc_gather_e_segsum():
    return pl.kernel(
        _sc_gather_e_segsum_body,
        out_type=(jax.ShapeDtypeStruct((NC, NPAD, H), jnp.float32),
                  jax.ShapeDtypeStruct((NC, NPAD, H), jnp.float32)),
        mesh=_mesh(),
        compiler_params=pltpu.CompilerParams(use_tc_tiling_on_sc=False),
        scratch_types=[
            pltpu.VMEM((NBLK, BK), jnp.int32),
            pltpu.VMEM((NBLK, BK), jnp.int32),
            [pltpu.VMEM((BK, H), jnp.float32)] * NB,
            [pltpu.VMEM((BK, H), jnp.float32)] * NB,
            pltpu.VMEM((BK, H), jnp.float32),
            pltpu.VMEM_SHARED((NPAD, H), jnp.float32),
            pltpu.VMEM_SHARED((NPAD, H), jnp.float32),
            [pltpu.SemaphoreType.DMA] * NB,
            [pltpu.SemaphoreType.DMA] * NB,
        ],
    )


def _sc_gather_e_segsum_body(table, ev, srcR, dstR, outP, outQ, sidx, didx,
                             rows, erow, zbuf, accP, accQ, gsem, esem):
    cid = lax.axis_index("c")
    sid = lax.axis_index("s")
    wid = sid * NC + cid
    _zero_vmem(zbuf, BK)

    @pl.loop(0, RPS // BK)
    def _(b):
        pltpu.sync_copy(zbuf, accP.at[pl.ds(sid * RPS + b * BK, BK)])
        pltpu.sync_copy(zbuf, accQ.at[pl.ds(sid * RPS + b * BK, BK)])

    pltpu.sync_copy(srcR.at[wid], sidx)
    pltpu.sync_copy(dstR.at[wid], didx)
    for b in range(NB):
        pltpu.async_copy(table.at[sidx.at[b]], rows[b], gsem[b])
        pltpu.async_copy(ev.at[pl.ds(wid * EPW + b * BK, BK)], erow[b], esem[b])
    plsc.subcore_barrier()

    @pl.loop(0, NBLK, step=NB)
    def _(g):
        for b in range(NB):
            j = g + b
            pltpu.make_async_copy(table.at[sidx.at[j]], rows[b], gsem[b]).wait()
            pltpu.sync_copy(rows[b], accP.at[didx.at[j]], add=True)
            pltpu.make_async_copy(
                ev.at[pl.ds(wid * EPW + j * BK, BK)], erow[b], esem[b]).wait()
            pltpu.sync_copy(erow[b], accQ.at[didx.at[j]], add=True)

            @pl.when(j + NB < NBLK)
            def _():
                pltpu.async_copy(table.at[sidx.at[j + NB]], rows[b], gsem[b])
                pltpu.async_copy(
                    ev.at[pl.ds(wid * EPW + (j + NB) * BK, BK)], erow[b], esem[b])

    plsc.subcore_barrier()
    pltpu.sync_copy(accP.at[pl.ds(sid * RPS, RPS)],
                    outP.at[cid].at[pl.ds(sid * RPS, RPS)])
    pltpu.sync_copy(accQ.at[pl.ds(sid * RPS, RPS)],
                    outQ.at[cid].at[pl.ds(sid * RPS, RPS)])


@functools.lru_cache(maxsize=None)
def _sc_gather_pair():
    return pl.kernel(
        _sc_gather_pair_body,
        out_type=jax.ShapeDtypeStruct((E, H), jnp.float32),
        mesh=_mesh(),
        compiler_params=pltpu.CompilerParams(use_tc_tiling_on_sc=False),
        scratch_types=[
            pltpu.VMEM((NBLK, BK), jnp.int32),
            pltpu.VMEM((NBLK, BK), jnp.int32),
            [pltpu.VMEM((BK, H), jnp.float32)] * NB,
            [pltpu.VMEM((BK, H), jnp.float32)] * NB,
            [pltpu.VMEM((BK, H), jnp.float32)] * NB,
            [pltpu.SemaphoreType.DMA] * NB,
            [pltpu.SemaphoreType.DMA] * NB,
            [pltpu.SemaphoreType.DMA] * NB,
        ],
    )


def _sc_gather_pair_body(table, srcR, dstR, out, sidx, didx, r1, r2, ob,
                         sem1, sem2, semo):
    cid = lax.axis_index("c")
    sid = lax.axis_index("s")
    wid = sid * NC + cid
    pltpu.sync_copy(srcR.at[wid], sidx)
    pltpu.sync_copy(dstR.at[wid], didx)
    for b in range(NB):
        pltpu.async_copy(table.at[sidx.at[b]], r1[b], sem1[b])
        pltpu.async_copy(table.at[didx.at[b]], r2[b], sem2[b])

    @pl.loop(0, NBLK, step=NB)
    def _(g):
        for b in range(NB):
            j = g + b
            pltpu.make_async_copy(table.at[sidx.at[j]], r1[b], sem1[b]).wait()
            pltpu.make_async_copy(table.at[didx.at[j]], r2[b], sem2[b]).wait()

            @pl.when(j >= NB)
            def _():
                pltpu.make_async_copy(
                    ob[b], out.at[pl.ds(wid * EPW + (j - NB) * BK, BK)],
                    semo[b]).wait()

            @pl.loop(0, BK)
            def _(r):
                for c in range(H // 16):
                    sl = pl.ds(16 * c, 16)
                    ob[b][r, sl] = r1[b][r, sl] + r2[b][r, sl]

            @pl.when(j + NB < NBLK)
            def _():
                pltpu.async_copy(table.at[sidx.at[j + NB]], r1[b], sem1[b])
                pltpu.async_copy(table.at[didx.at[j + NB]], r2[b], sem2[b])

            pltpu.async_copy(
                ob[b], out.at[pl.ds(wid * EPW + j * BK, BK)], semo[b])

    for b in range(NB):
        pltpu.make_async_copy(
            ob[b], out.at[pl.ds(wid * EPW, BK)], semo[b]).wait()


def _ln_relu(y):
    mu = jnp.mean(y, axis=-1, keepdims=True)
    var = jnp.mean((y - mu) ** 2, axis=-1, keepdims=True)
    return jnp.maximum((y - mu) / jnp.sqrt(var + 1e-5), 0.0)


BN = 640
BE2 = 2000


def _t_init_nodes(x, Wn):
    def body(x_ref, w_ref, o_ref):
        o_ref[...] = _ln_relu(jnp.dot(x_ref[...], w_ref[...],
                                      preferred_element_type=jnp.float32))
    return pl.pallas_call(
        body,
        grid=(NPAD // BN,),
        in_specs=[pl.BlockSpec((BN, DF), lambda i: (i, 0)),
                  pl.BlockSpec((DF, H), lambda i: (0, 0))],
        out_specs=pl.BlockSpec((BN, H), lambda i: (i, 0)),
        out_shape=jax.ShapeDtypeStruct((NPAD, H), jnp.float32),
    )(x, Wn)


def _t_edge_init(eaR, We):
    def body(a_ref, b_ref, w_ref, o_ref):
        ya = _ln_relu(jnp.dot(a_ref[...], w_ref[...],
                              preferred_element_type=jnp.float32))
        yb = _ln_relu(jnp.dot(b_ref[...], w_ref[...],
                              preferred_element_type=jnp.float32))
        o_ref[...] = jnp.concatenate([ya, yb], axis=1)
    return pl.pallas_call(
        body,
        grid=(E2 // BE2,),
        in_specs=[pl.BlockSpec((BE2, DE), lambda i: (i, 0)),
                  pl.BlockSpec((BE2, DE), lambda i: (i + E2 // BE2, 0)),
                  pl.BlockSpec((DE, H), lambda i: (0, 0))],
        out_specs=pl.BlockSpec((BE2, 2 * H), lambda i: (i, 0)),
        out_shape=jax.ShapeDtypeStruct((E2, 2 * H), jnp.float32),
    )(eaR, eaR, We)


def _t_level(n, P, Q, W, W_next=None):
    two_out = W_next is not None

    def body(*refs):
        if two_out:
            n_ref, p_ref, q_ref, w_ref, wn_ref, o_ref, t_ref = refs
        else:
            n_ref, p_ref, q_ref, w_ref, o_ref = refs
        s = (p_ref[0] + p_ref[1] + q_ref[0] + q_ref[1])
        y = _ln_relu(n_ref[...] + jnp.dot(s, w_ref[...],
                                          preferred_element_type=jnp.float32))
        o_ref[...] = y
        if two_out:
            t_ref[...] = jnp.dot(y, wn_ref[...],
                                 preferred_element_type=jnp.float32)

    in_specs = [pl.BlockSpec((BN, H), lambda i: (i, 0)),
                pl.BlockSpec((NC, BN, H), lambda i: (0, i, 0)),
                pl.BlockSpec((NC, BN, H), lambda i: (0, i, 0)),
                pl.BlockSpec((H, H), lambda i: (0, 0))]
    out_spec = pl.BlockSpec((BN, H), lambda i: (i, 0))
    shp = jax.ShapeDtypeStruct((NPAD, H), jnp.float32)
    if two_out:
        return pl.pallas_call(
            body, grid=(NPAD // BN,),
            in_specs=in_specs + [pl.BlockSpec((H, H), lambda i: (0, 0))],
            out_specs=(out_spec, out_spec), out_shape=(shp, shp),
        )(n, P, Q, W, W_next)
    return pl.pallas_call(
        body, grid=(NPAD // BN,),
        in_specs=in_specs, out_specs=out_spec, out_shape=shp,
    )(n, P, Q, W)


def _t_segnode(top, nbR):
    def body(t_ref, nb_ref, o_ref):
        i = pl.program_id(0)
        oh = (lax.broadcasted_iota(jnp.int32, (B, BN), 0)
              == nb_ref[0, 0, :][None, :]).astype(jnp.float32)
        part = jnp.dot(oh, t_ref[...], preferred_element_type=jnp.float32)

        @pl.when(i == 0)
        def _():
            o_ref[...] = jnp.zeros_like(o_ref)

        o_ref[...] += part

    return pl.pallas_call(
        body, grid=(NPAD // BN,),
        in_specs=[pl.BlockSpec((BN, H), lambda i: (i, 0)),
                  pl.BlockSpec((1, 1, BN), lambda i: (i, 0, 0))],
        out_specs=pl.BlockSpec((B, H), lambda i: (0, 0)),
        out_shape=jax.ShapeDtypeStruct((B, H), jnp.float32),
    )(top, nbR)


def _t_assign(top, nbR, node_agg, W_as):
    def body(t_ref, nb_ref, na_ref, w_ref, o_ref):
        M = jnp.dot(na_ref[...], w_ref[...], preferred_element_type=jnp.float32)
        oh = (nb_ref[0, 0, :][:, None]
              == lax.broadcasted_iota(jnp.int32, (BN, B), 1)).astype(jnp.float32)
        ctx = jnp.dot(oh, M, preferred_element_type=jnp.float32)
        o_ref[...] = _ln_relu(t_ref[...] + ctx)

    return pl.pallas_call(
        body, grid=(NPAD // BN,),
        in_specs=[pl.BlockSpec((BN, H), lambda i: (i, 0)),
                  pl.BlockSpec((1, 1, BN), lambda i: (i, 0, 0)),
                  pl.BlockSpec((B, H), lambda i: (0, 0)),
                  pl.BlockSpec((H, H), lambda i: (0, 0))],
        out_specs=pl.BlockSpec((BN, H), lambda i: (i, 0)),
        out_shape=jax.ShapeDtypeStruct((NPAD, H), jnp.float32),
    )(top, nbR, node_agg, W_as)


def _t_edge_update(e_prev, g, W):
    def body(e_ref, g_ref, w_ref, o_ref):
        e = e_ref[...]
        gg = g_ref[...]
        w = w_ref[...]
        ya = _ln_relu(jnp.dot(e[:, :H], w, preferred_element_type=jnp.float32)
                      + gg[:, :H])
        yb = _ln_relu(jnp.dot(e[:, H:], w, preferred_element_type=jnp.float32)
                      + gg[:, H:])
        o_ref[...] = jnp.concatenate([ya, yb], axis=1)
    return pl.pallas_call(
        body, grid=(E2 // BE2,),
        in_specs=[pl.BlockSpec((BE2, 2 * H), lambda i: (i, 0)),
                  pl.BlockSpec((BE2, 2 * H), lambda i: (i, 0)),
                  pl.BlockSpec((H, H), lambda i: (0, 0))],
        out_specs=pl.BlockSpec((BE2, 2 * H), lambda i: (i, 0)),
        out_shape=jax.ShapeDtypeStruct((E2, 2 * H), jnp.float32),
    )(e_prev, g, W)


def _t_edge_final(e_prev, g, W, ebR):
    def body(e_ref, g_ref, w_ref, eba_ref, ebb_ref, o_ref):
        i = pl.program_id(0)
        e = e_ref[...]
        gg = g_ref[...]
        w = w_ref[...]
        ya = _ln_relu(jnp.dot(e[:, :H], w, preferred_element_type=jnp.float32)
                      + gg[:, :H])
        yb = _ln_relu(jnp.dot(e[:, H:], w, preferred_element_type=jnp.float32)
                      + gg[:, H:])
        oha = (lax.broadcasted_iota(jnp.int32, (B, BE2), 0)
               == eba_ref[0, 0, :][None, :]).astype(jnp.float32)
        ohb = (lax.broadcasted_iota(jnp.int32, (B, BE2), 0)
               == ebb_ref[0, 0, :][None, :]).astype(jnp.float32)
        part = (jnp.dot(oha, ya, preferred_element_type=jnp.float32)
                + jnp.dot(ohb, yb, preferred_element_type=jnp.float32))

        @pl.when(i == 0)
        def _():
            o_ref[...] = jnp.zeros_like(o_ref)

        o_ref[...] += part

    return pl.pallas_call(
        body, grid=(E2 // BE2,),
        in_specs=[pl.BlockSpec((BE2, 2 * H), lambda i: (i, 0)),
                  pl.BlockSpec((BE2, 2 * H), lambda i: (i, 0)),
                  pl.BlockSpec((H, H), lambda i: (0, 0)),
                  pl.BlockSpec((1, 1, BE2), lambda i: (i, 0, 0)),
                  pl.BlockSpec((1, 1, BE2), lambda i: (i + E2 // BE2, 0, 0))],
        out_specs=pl.BlockSpec((B, H), lambda i: (0, 0)),
        out_shape=jax.ShapeDtypeStruct((B, H), jnp.float32),
    )(e_prev, g, W, ebR, ebR)


def _t_final(na, ea, W1, W2, b1, W_out, b_out):
    def body(na_ref, ea_ref, w1_ref, w2_ref, b1_ref, wo_ref, bo_ref, o_ref):
        h = _ln_relu(jnp.dot(na_ref[...], w1_ref[...],
                             preferred_element_type=jnp.float32)
                     + jnp.dot(ea_ref[...], w2_ref[...],
                               preferred_element_type=jnp.float32)
                     + b1_ref[...])
        o_ref[...] = jnp.dot(h, wo_ref[...],
                             preferred_element_type=jnp.float32) + bo_ref[...]

    spec = pl.BlockSpec((B, H), lambda: (0, 0))
    wspec = pl.BlockSpec((H, H), lambda: (0, 0))
    return pl.pallas_call(
        body,
        in_specs=[spec, spec, wspec, wspec,
                  pl.BlockSpec((1, H), lambda: (0, 0)),
                  pl.BlockSpec((H, H), lambda: (0, 0)),
                  pl.BlockSpec((1, H), lambda: (0, 0))],
        out_specs=pl.BlockSpec((B, H), lambda: (0, 0)),
        out_shape=jax.ShapeDtypeStruct((B, H), jnp.float32),
    )(na, ea, W1, W2, b1, W_out, b_out)


def kernel(x, edge_attr, Wn, We, W_bu, W_eu, W_assign, W1, W2, b1, W_out, b_out,
           edge_index, node_batch, edge_batch):
    src = edge_index[0].astype(jnp.int32)
    dst = edge_index[1].astype(jnp.int32)
    src_p = jnp.stack([src[:E2], src[E2:]], axis=1).reshape(-1)
    dst_p = jnp.stack([dst[:E2], dst[E2:]], axis=1).reshape(-1)
    srcR = src_p.reshape(NW, NBLK, BK)
    dstR = dst_p.reshape(NW, NBLK, BK)
    xp = jnp.pad(x, ((0, NPAD - N), (0, 0)))
    eaR = edge_attr
    nbR = jnp.pad(node_batch.astype(jnp.int32), (0, NPAD - N),
                  constant_values=B).reshape(NPAD // BN, 1, BN)
    ebR = edge_batch.astype(jnp.int32).reshape(E // BE2, 1, BE2)
    W1p = jnp.pad(W_out, ((0, 0), (0, H - 1)))
    b1p = b1.reshape(1, H)
    bop = jnp.pad(b_out, (0, H - 1)).reshape(1, H)

    n = _t_init_nodes(xp, Wn)
    e0 = _t_edge_init(eaR, We)

    Q = _sc_scatter_segsum()(jnp.reshape(e0, (E, H)), dstR)

    for lvl in range(HEIGHT):
        P = _sc_gather_segsum()(n, srcR, dstR)
        if lvl == HEIGHT - 1:
            n, t0 = _t_level(n, P, Q, W_bu[0, lvl], W_eu[0])
        else:
            n = _t_level(n, P, Q, W_bu[0, lvl])
    top0 = n

    g0 = jnp.reshape(_sc_gather_pair()(t0, srcR, dstR), (E2, 2 * H))
    e1 = _t_edge_update(e0, g0, W_eu[0])

    na0 = _t_segnode(top0, nbR)
    n = _t_assign(top0, nbR, na0, W_assign[0])

    Q1 = _sc_scatter_segsum()(jnp.reshape(e1, (E, H)), dstR)

    for lvl in range(HEIGHT):
        P = _sc_gather_segsum()(n, srcR, dstR)
        if lvl == HEIGHT - 1:
            n, t1 = _t_level(n, P, Q1, W_bu[1, lvl], W_eu[1])
        else:
            n = _t_level(n, P, Q1, W_bu[1, lvl])
    top1 = n

    g1 = jnp.reshape(_sc_gather_pair()(t1, srcR, dstR), (E2, 2 * H))
    edge_agg = _t_edge_final(e1, g1, W_eu[1], ebR)
    naf = _t_segnode(top1, nbR)

    out = _t_final(naf, edge_agg, W1, W2, b1p, W1p, bop)
    return out[:, :1]


# speedup vs baseline: 1.5615x; 1.0757x over previous
import functools

import jax
import jax.numpy as jnp
from jax import lax
from jax.experimental import pallas as pl
from jax.experimental.pallas import tpu as pltpu
from jax.experimental.pallas import tpu_sc as plsc

N = 10000
NPAD = 10240
E = 320000
E2 = E // 2
DF = 128
DE = 16
H = 64
B = 64
HEIGHT = 3

NC = 2
NS = 16
NW = NC * NS
EPW = E // NW
BK = 80
NBLK = EPW // BK
RPS = NPAD // NS
NPAD2 = NPAD // 2

@functools.lru_cache(maxsize=None)
def _mesh():
    return plsc.VectorSubcoreMesh(
        core_axis_name="c", subcore_axis_name="s", num_cores=NC, num_subcores=NS)


def _zero_vmem(buf, rows):
    z = jnp.zeros((16,), jnp.float32)

    @pl.loop(0, rows)
    def _(r):
        for c in range(H // 16):
            buf[r, pl.ds(16 * c, 16)] = z


NB = 5


@functools.lru_cache(maxsize=None)
def _sc_gather_segsum():
    return pl.kernel(
        _sc_gather_segsum_body,
        out_type=jax.ShapeDtypeStruct((NC, NPAD, H), jnp.float32),
        mesh=_mesh(),
        compiler_params=pltpu.CompilerParams(use_tc_tiling_on_sc=False),
        scratch_types=[
            pltpu.VMEM((NBLK, BK), jnp.int32),
            pltpu.VMEM((NBLK, BK), jnp.int32),
            [pltpu.VMEM((BK, H), jnp.float32)] * NB,
            pltpu.VMEM((BK, H), jnp.float32),
            pltpu.VMEM_SHARED((NPAD, H), jnp.float32),
            [pltpu.SemaphoreType.DMA] * NB,
        ],
    )


def _sc_gather_segsum_body(table, srcR, dstR, out, sidx, didx, rows, zbuf, acc,
                           sems):
    cid = lax.axis_index("c")
    sid = lax.axis_index("s")
    wid = sid * NC + cid
    _zero_vmem(zbuf, BK)

    @pl.loop(0, RPS // BK)
    def _(b):
        pltpu.sync_copy(zbuf, acc.at[pl.ds(sid * RPS + b * BK, BK)])

    pltpu.sync_copy(srcR.at[wid], sidx)
    pltpu.sync_copy(dstR.at[wid], didx)
    for b in range(NB):
        pltpu.async_copy(table.at[sidx.at[b]], rows[b], sems[b])
    plsc.subcore_barrier()

    @pl.loop(0, NBLK, step=NB)
    def _(g):
        for b in range(NB):
            j = g + b
            pltpu.make_async_copy(table.at[sidx.at[j]], rows[b], sems[b]).wait()
            pltpu.sync_copy(rows[b], acc.at[didx.at[j]], add=True)

            @pl.when(j + NB < NBLK)
            def _():
                pltpu.async_copy(table.at[sidx.at[j + NB]], rows[b], sems[b])

    plsc.subcore_barrier()
    pltpu.sync_copy(acc.at[pl.ds(sid * RPS, RPS)],
                    out.at[cid].at[pl.ds(sid * RPS, RPS)])


@functools.lru_cache(maxsize=None)
def _sc_scatter_segsum():
    return pl.kernel(
        _sc_scatter_segsum_body,
        out_type=jax.ShapeDtypeStruct((NC, NPAD, H), jnp.float32),
        mesh=_mesh(),
        compiler_params=pltpu.CompilerParams(use_tc_tiling_on_sc=False),
        scratch_types=[
            pltpu.VMEM((NBLK, BK), jnp.int32),
            [pltpu.VMEM((BK, H), jnp.float32)] * NB,
            pltpu.VMEM((BK, H), jnp.float32),
            pltpu.VMEM_SHARED((NPAD, H), jnp.float32),
            [pltpu.SemaphoreType.DMA] * NB,
        ],
    )


def _sc_scatter_segsum_body(ev, dstR, out, didx, rows, zbuf, acc, sems):
    cid = lax.axis_index("c")
    sid = lax.axis_index("s")
    wid = sid * NC + cid
    _zero_vmem(zbuf, BK)

    @pl.loop(0, RPS // BK)
    def _(b):
        pltpu.sync_copy(zbuf, acc.at[pl.ds(sid * RPS + b * BK, BK)])

    pltpu.sync_copy(dstR.at[wid], didx)
    for b in range(NB):
        pltpu.async_copy(ev.at[pl.ds(wid * EPW + b * BK, BK)], rows[b], sems[b])
    plsc.subcore_barrier()

    @pl.loop(0, NBLK, step=NB)
    def _(g):
        for b in range(NB):
            j = g + b
            pltpu.make_async_copy(
                ev.at[pl.ds(wid * EPW + j * BK, BK)], rows[b], sems[b]).wait()
            pltpu.sync_copy(rows[b], acc.at[didx.at[j]], add=True)

            @pl.when(j + NB < NBLK)
            def _():
                pltpu.async_copy(
                    ev.at[pl.ds(wid * EPW + (j + NB) * BK, BK)], rows[b], sems[b])

    plsc.subcore_barrier()
    pltpu.sync_copy(acc.at[pl.ds(sid * RPS, RPS)],
                    out.at[cid].at[pl.ds(sid * RPS, RPS)])


@functools.lru_cache(maxsize=None)
def _sc_gather_e_segsum():
    return pl.kernel(
        _sc_gather_e_segsum_body,
        out_type=(jax.ShapeDtypeStruct((NC, NPAD, H), jnp.float32),
                  jax.ShapeDtypeStruct((NC, NPAD, H), jnp.float32)),
        mesh=_mesh(),
        compiler_params=pltpu.CompilerParams(use_tc_tiling_on_sc=False),
        scratch_types=[
            pltpu.VMEM((NBLK, BK), jnp.int32),
            pltpu.VMEM((NBLK, BK), jnp.int32),
            [pltpu.VMEM((BK, H), jnp.float32)] * NB,
            [pltpu.VMEM((BK, H), jnp.float32)] * NB,
            pltpu.VMEM((BK, H), jnp.float32),
            pltpu.VMEM_SHARED((NPAD, H), jnp.float32),
            pltpu.VMEM_SHARED((NPAD, H), jnp.float32),
            [pltpu.SemaphoreType.DMA] * NB,
            [pltpu.SemaphoreType.DMA] * NB,
        ],
    )


def _sc_gather_e_segsum_body(table, ev, srcR, dstR, outP, outQ, sidx, didx,
                             rows, erow, zbuf, accP, accQ, gsem, esem):
    cid = lax.axis_index("c")
    sid = lax.axis_index("s")
    wid = sid * NC + cid
    _zero_vmem(zbuf, BK)

    @pl.loop(0, RPS // BK)
    def _(b):
        pltpu.sync_copy(zbuf, accP.at[pl.ds(sid * RPS + b * BK, BK)])
        pltpu.sync_copy(zbuf, accQ.at[pl.ds(sid * RPS + b * BK, BK)])

    pltpu.sync_copy(srcR.at[wid], sidx)
    pltpu.sync_copy(dstR.at[wid], didx)
    for b in range(NB):
        pltpu.async_copy(table.at[sidx.at[b]], rows[b], gsem[b])
        pltpu.async_copy(ev.at[pl.ds(wid * EPW + b * BK, BK)], erow[b], esem[b])
    plsc.subcore_barrier()

    @pl.loop(0, NBLK, step=NB)
    def _(g):
        for b in range(NB):
            j = g + b
            pltpu.make_async_copy(table.at[sidx.at[j]], rows[b], gsem[b]).wait()
            pltpu.sync_copy(rows[b], accP.at[didx.at[j]], add=True)
            pltpu.make_async_copy(
                ev.at[pl.ds(wid * EPW + j * BK, BK)], erow[b], esem[b]).wait()
            pltpu.sync_copy(erow[b], accQ.at[didx.at[j]], add=True)

            @pl.when(j + NB < NBLK)
            def _():
                pltpu.async_copy(table.at[sidx.at[j + NB]], rows[b], gsem[b])
                pltpu.async_copy(
                    ev.at[pl.ds(wid * EPW + (j + NB) * BK, BK)], erow[b], esem[b])

    plsc.subcore_barrier()
    pltpu.sync_copy(accP.at[pl.ds(sid * RPS, RPS)],
                    outP.at[cid].at[pl.ds(sid * RPS, RPS)])
    pltpu.sync_copy(accQ.at[pl.ds(sid * RPS, RPS)],
                    outQ.at[cid].at[pl.ds(sid * RPS, RPS)])


@functools.lru_cache(maxsize=None)
def _sc_gather_pair():
    return pl.kernel(
        _sc_gather_pair_body,
        out_type=jax.ShapeDtypeStruct((E, H), jnp.float32),
        mesh=_mesh(),
        compiler_params=pltpu.CompilerParams(use_tc_tiling_on_sc=False),
        scratch_types=[
            pltpu.VMEM((NBLK, BK), jnp.int32),
            pltpu.VMEM((NBLK, BK), jnp.int32),
            [pltpu.VMEM((BK, H), jnp.float32)] * NB,
            [pltpu.VMEM((BK, H), jnp.float32)] * NB,
            [pltpu.VMEM((BK, H), jnp.float32)] * NB,
            [pltpu.SemaphoreType.DMA] * NB,
            [pltpu.SemaphoreType.DMA] * NB,
            [pltpu.SemaphoreType.DMA] * NB,
        ],
    )


def _sc_gather_pair_body(table, srcR, dstR, out, sidx, didx, r1, r2, ob,
                         sem1, sem2, semo):
    cid = lax.axis_index("c")
    sid = lax.axis_index("s")
    wid = sid * NC + cid
    pltpu.sync_copy(srcR.at[wid], sidx)
    pltpu.sync_copy(dstR.at[wid], didx)
    for b in range(NB):
        pltpu.async_copy(table.at[sidx.at[b]], r1[b], sem1[b])
        pltpu.async_copy(table.at[didx.at[b]], r2[b], sem2[b])

    @pl.loop(0, NBLK, step=NB)
    def _(g):
        for b in range(NB):
            j = g + b
            pltpu.make_async_copy(table.at[sidx.at[j]], r1[b], sem1[b]).wait()
            pltpu.make_async_copy(table.at[didx.at[j]], r2[b], sem2[b]).wait()

            @pl.when(j >= NB)
            def _():
                pltpu.make_async_copy(
                    ob[b], out.at[pl.ds(wid * EPW + (j - NB) * BK, BK)],
                    semo[b]).wait()

            @pl.loop(0, BK)
            def _(r):
                for c in range(H // 16):
                    sl = pl.ds(16 * c, 16)
                    ob[b][r, sl] = r1[b][r, sl] + r2[b][r, sl]

            @pl.when(j + NB < NBLK)
            def _():
                pltpu.async_copy(table.at[sidx.at[j + NB]], r1[b], sem1[b])
                pltpu.async_copy(table.at[didx.at[j + NB]], r2[b], sem2[b])

            pltpu.async_copy(
                ob[b], out.at[pl.ds(wid * EPW + j * BK, BK)], semo[b])

    for b in range(NB):
        pltpu.make_async_copy(
            ob[b], out.at[pl.ds(wid * EPW, BK)], semo[b]).wait()


def _ln_relu(y):
    mu = jnp.mean(y, axis=-1, keepdims=True)
    var = jnp.mean((y - mu) ** 2, axis=-1, keepdims=True)
    return jnp.maximum((y - mu) / jnp.sqrt(var + 1e-5), 0.0)


BN = 640
BE2 = 2000


def _t_init_nodes(x, Wn):
    def body(a_ref, b_ref, w_ref, o_ref):
        ya = _ln_relu(jnp.dot(a_ref[...], w_ref[...],
                              preferred_element_type=jnp.float32))
        yb = _ln_relu(jnp.dot(b_ref[...], w_ref[...],
                              preferred_element_type=jnp.float32))
        o_ref[...] = jnp.concatenate([ya, yb], axis=1)
    return pl.pallas_call(
        body,
        grid=(NPAD2 // BN,),
        in_specs=[pl.BlockSpec((BN, DF), lambda i: (i, 0)),
                  pl.BlockSpec((BN, DF), lambda i: (i + NPAD2 // BN, 0)),
                  pl.BlockSpec((DF, H), lambda i: (0, 0))],
        out_specs=pl.BlockSpec((BN, 2 * H), lambda i: (i, 0)),
        out_shape=jax.ShapeDtypeStruct((NPAD2, 2 * H), jnp.float32),
    )(x, x, Wn)


def _t_edge_init(eaR, We):
    def body(a_ref, b_ref, w_ref, o_ref):
        ya = _ln_relu(jnp.dot(a_ref[...], w_ref[...],
                              preferred_element_type=jnp.float32))
        yb = _ln_relu(jnp.dot(b_ref[...], w_ref[...],
                              preferred_element_type=jnp.float32))
        o_ref[...] = jnp.concatenate([ya, yb], axis=1)
    return pl.pallas_call(
        body,
        grid=(E2 // BE2,),
        in_specs=[pl.BlockSpec((BE2, DE), lambda i: (i, 0)),
                  pl.BlockSpec((BE2, DE), lambda i: (i + E2 // BE2, 0)),
                  pl.BlockSpec((DE, H), lambda i: (0, 0))],
        out_specs=pl.BlockSpec((BE2, 2 * H), lambda i: (i, 0)),
        out_shape=jax.ShapeDtypeStruct((E2, 2 * H), jnp.float32),
    )(eaR, eaR, We)


def _t_level(n, P, Q, W, W_next=None):
    two_out = W_next is not None

    def body(*refs):
        if two_out:
            n_ref, p_ref, q_ref, w_ref, wn_ref, o_ref, t_ref = refs
        else:
            n_ref, p_ref, q_ref, w_ref, o_ref = refs
        s = (p_ref[0] + p_ref[1] + q_ref[0] + q_ref[1])
        nn = n_ref[...]
        w = w_ref[...]
        ya = _ln_relu(nn[:, :H] + jnp.dot(s[:, :H], w,
                                          preferred_element_type=jnp.float32))
        yb = _ln_relu(nn[:, H:] + jnp.dot(s[:, H:], w,
                                          preferred_element_type=jnp.float32))
        o_ref[...] = jnp.concatenate([ya, yb], axis=1)
        if two_out:
            wn = wn_ref[...]
            t_ref[...] = jnp.concatenate(
                [jnp.dot(ya, wn, preferred_element_type=jnp.float32),
                 jnp.dot(yb, wn, preferred_element_type=jnp.float32)], axis=1)

    in_specs = [pl.BlockSpec((BN, 2 * H), lambda i: (i, 0)),
                pl.BlockSpec((NC, BN, 2 * H), lambda i: (0, i, 0)),
                pl.BlockSpec((NC, BN, 2 * H), lambda i: (0, i, 0)),
                pl.BlockSpec((H, H), lambda i: (0, 0))]
    out_spec = pl.BlockSpec((BN, 2 * H), lambda i: (i, 0))
    shp = jax.ShapeDtypeStruct((NPAD2, 2 * H), jnp.float32)
    if two_out:
        return pl.pallas_call(
            body, grid=(NPAD2 // BN,),
            in_specs=in_specs + [pl.BlockSpec((H, H), lambda i: (0, 0))],
            out_specs=(out_spec, out_spec), out_shape=(shp, shp),
        )(n, P, Q, W, W_next)
    return pl.pallas_call(
        body, grid=(NPAD2 // BN,),
        in_specs=in_specs, out_specs=out_spec, out_shape=shp,
    )(n, P, Q, W)


def _t_segnode(top, nbR):
    def body(t_ref, nba_ref, nbb_ref, o_ref):
        i = pl.program_id(0)
        t = t_ref[...]
        oha = (lax.broadcasted_iota(jnp.int32, (B, BN), 0)
               == nba_ref[0, 0, :][None, :]).astype(jnp.float32)
        ohb = (lax.broadcasted_iota(jnp.int32, (B, BN), 0)
               == nbb_ref[0, 0, :][None, :]).astype(jnp.float32)
        part = (jnp.dot(oha, t[:, :H], preferred_element_type=jnp.float32)
                + jnp.dot(ohb, t[:, H:], preferred_element_type=jnp.float32))

        @pl.when(i == 0)
        def _():
            o_ref[...] = jnp.zeros_like(o_ref)

        o_ref[...] += part

    return pl.pallas_call(
        body, grid=(NPAD2 // BN,),
        in_specs=[pl.BlockSpec((BN, 2 * H), lambda i: (i, 0)),
                  pl.BlockSpec((1, 1, BN), lambda i: (i, 0, 0)),
                  pl.BlockSpec((1, 1, BN), lambda i: (i + NPAD2 // BN, 0, 0))],
        out_specs=pl.BlockSpec((B, H), lambda i: (0, 0)),
        out_shape=jax.ShapeDtypeStruct((B, H), jnp.float32),
    )(top, nbR, nbR)


def _t_assign(top, nbR, node_agg, W_as):
    def body(t_ref, nba_ref, nbb_ref, na_ref, w_ref, o_ref):
        M = jnp.dot(na_ref[...], w_ref[...], preferred_element_type=jnp.float32)
        t = t_ref[...]
        oha = (nba_ref[0, 0, :][:, None]
               == lax.broadcasted_iota(jnp.int32, (BN, B), 1)).astype(jnp.float32)
        ohb = (nbb_ref[0, 0, :][:, None]
               == lax.broadcasted_iota(jnp.int32, (BN, B), 1)).astype(jnp.float32)
        ya = _ln_relu(t[:, :H] + jnp.dot(oha, M,
                                         preferred_element_type=jnp.float32))
        yb = _ln_relu(t[:, H:] + jnp.dot(ohb, M,
                                         preferred_element_type=jnp.float32))
        o_ref[...] = jnp.concatenate([ya, yb], axis=1)

    return pl.pallas_call(
        body, grid=(NPAD2 // BN,),
        in_specs=[pl.BlockSpec((BN, 2 * H), lambda i: (i, 0)),
                  pl.BlockSpec((1, 1, BN), lambda i: (i, 0, 0)),
                  pl.BlockSpec((1, 1, BN), lambda i: (i + NPAD2 // BN, 0, 0)),
                  pl.BlockSpec((B, H), lambda i: (0, 0)),
                  pl.BlockSpec((H, H), lambda i: (0, 0))],
        out_specs=pl.BlockSpec((BN, 2 * H), lambda i: (i, 0)),
        out_shape=jax.ShapeDtypeStruct((NPAD2, 2 * H), jnp.float32),
    )(top, nbR, nbR, node_agg, W_as)


def _t_edge_update(e_prev, g, W):
    def body(e_ref, g_ref, w_ref, o_ref):
        e = e_ref[...]
        gg = g_ref[...]
        w = w_ref[...]
        ya = _ln_relu(jnp.dot(e[:, :H], w, preferred_element_type=jnp.float32)
                      + gg[:, :H])
        yb = _ln_relu(jnp.dot(e[:, H:], w, preferred_element_type=jnp.float32)
                      + gg[:, H:])
        o_ref[...] = jnp.concatenate([ya, yb], axis=1)
    return pl.pallas_call(
        body, grid=(E2 // BE2,),
        in_specs=[pl.BlockSpec((BE2, 2 * H), lambda i: (i, 0)),
                  pl.BlockSpec((BE2, 2 * H), lambda i: (i, 0)),
                  pl.BlockSpec((H, H), lambda i: (0, 0))],
        out_specs=pl.BlockSpec((BE2, 2 * H), lambda i: (i, 0)),
        out_shape=jax.ShapeDtypeStruct((E2, 2 * H), jnp.float32),
    )(e_prev, g, W)


def _t_edge_final(e_prev, g, W, ebR):
    def body(e_ref, g_ref, w_ref, eba_ref, ebb_ref, o_ref):
        i = pl.program_id(0)
        e = e_ref[...]
        gg = g_ref[...]
        w = w_ref[...]
        ya = _ln_relu(jnp.dot(e[:, :H], w, preferred_element_type=jnp.float32)
                      + gg[:, :H])
        yb = _ln_relu(jnp.dot(e[:, H:], w, preferred_element_type=jnp.float32)
                      + gg[:, H:])
        oha = (lax.broadcasted_iota(jnp.int32, (B, BE2), 0)
               == eba_ref[0, 0, :][None, :]).astype(jnp.float32)
        ohb = (lax.broadcasted_iota(jnp.int32, (B, BE2), 0)
               == ebb_ref[0, 0, :][None, :]).astype(jnp.float32)
        part = (jnp.dot(oha, ya, preferred_element_type=jnp.float32)
                + jnp.dot(ohb, yb, preferred_element_type=jnp.float32))

        @pl.when(i == 0)
        def _():
            o_ref[...] = jnp.zeros_like(o_ref)

        o_ref[...] += part

    return pl.pallas_call(
        body, grid=(E2 // BE2,),
        in_specs=[pl.BlockSpec((BE2, 2 * H), lambda i: (i, 0)),
                  pl.BlockSpec((BE2, 2 * H), lambda i: (i, 0)),
                  pl.BlockSpec((H, H), lambda i: (0, 0)),
                  pl.BlockSpec((1, 1, BE2), lambda i: (i, 0, 0)),
                  pl.BlockSpec((1, 1, BE2), lambda i: (i + E2 // BE2, 0, 0))],
        out_specs=pl.BlockSpec((B, H), lambda i: (0, 0)),
        out_shape=jax.ShapeDtypeStruct((B, H), jnp.float32),
    )(e_prev, g, W, ebR, ebR)


def _t_final(na, ea, W1, W2, b1, W_out, b_out):
    def body(na_ref, ea_ref, w1_ref, w2_ref, b1_ref, wo_ref, bo_ref, o_ref):
        h = _ln_relu(jnp.dot(na_ref[...], w1_ref[...],
                             preferred_element_type=jnp.float32)
                     + jnp.dot(ea_ref[...], w2_ref[...],
                               preferred_element_type=jnp.float32)
                     + b1_ref[...])
        o_ref[...] = jnp.dot(h, wo_ref[...],
                             preferred_element_type=jnp.float32) + bo_ref[...]

    spec = pl.BlockSpec((B, H), lambda: (0, 0))
    wspec = pl.BlockSpec((H, H), lambda: (0, 0))
    return pl.pallas_call(
        body,
        in_specs=[spec, spec, wspec, wspec,
                  pl.BlockSpec((1, H), lambda: (0, 0)),
                  pl.BlockSpec((H, H), lambda: (0, 0)),
                  pl.BlockSpec((1, H), lambda: (0, 0))],
        out_specs=pl.BlockSpec((B, H), lambda: (0, 0)),
        out_shape=jax.ShapeDtypeStruct((B, H), jnp.float32),
    )(na, ea, W1, W2, b1, W_out, b_out)


def kernel(x, edge_attr, Wn, We, W_bu, W_eu, W_assign, W1, W2, b1, W_out, b_out,
           edge_index, node_batch, edge_batch):
    src = edge_index[0].astype(jnp.int32)
    dst = edge_index[1].astype(jnp.int32)
    src = jnp.where(src < NPAD2, 2 * src, 2 * (src - NPAD2) + 1)
    dst = jnp.where(dst < NPAD2, 2 * dst, 2 * (dst - NPAD2) + 1)
    src_p = jnp.stack([src[:E2], src[E2:]], axis=1).reshape(-1)
    dst_p = jnp.stack([dst[:E2], dst[E2:]], axis=1).reshape(-1)
    srcR = src_p.reshape(NW, NBLK, BK)
    dstR = dst_p.reshape(NW, NBLK, BK)
    xp = jnp.pad(x, ((0, NPAD - N), (0, 0)))
    eaR = edge_attr
    nbR = jnp.pad(node_batch.astype(jnp.int32), (0, NPAD - N),
                  constant_values=B).reshape(NPAD // BN, 1, BN)
    flat = lambda a: jnp.reshape(a, (NPAD, H))
    flatPQ = lambda a: jnp.reshape(a, (NC, NPAD2, 2 * H))
    ebR = edge_batch.astype(jnp.int32).reshape(E // BE2, 1, BE2)
    W1p = jnp.pad(W_out, ((0, 0), (0, H - 1)))
    b1p = b1.reshape(1, H)
    bop = jnp.pad(b_out, (0, H - 1)).reshape(1, H)

    n = _t_init_nodes(xp, Wn)
    e0 = _t_edge_init(eaR, We)

    Q = flatPQ(_sc_scatter_segsum()(jnp.reshape(e0, (E, H)), dstR))

    for lvl in range(HEIGHT):
        P = flatPQ(_sc_gather_segsum()(flat(n), srcR, dstR))
        if lvl == HEIGHT - 1:
            n, t0 = _t_level(n, P, Q, W_bu[0, lvl], W_eu[0])
        else:
            n = _t_level(n, P, Q, W_bu[0, lvl])
    top0 = n

    g0 = jnp.reshape(_sc_gather_pair()(flat(t0), srcR, dstR), (E2, 2 * H))
    e1 = _t_edge_update(e0, g0, W_eu[0])

    na0 = _t_segnode(top0, nbR)
    n = _t_assign(top0, nbR, na0, W_assign[0])

    Q1 = flatPQ(_sc_scatter_segsum()(jnp.reshape(e1, (E, H)), dstR))

    for lvl in range(HEIGHT):
        P = flatPQ(_sc_gather_segsum()(flat(n), srcR, dstR))
        if lvl == HEIGHT - 1:
            n, t1 = _t_level(n, P, Q1, W_bu[1, lvl], W_eu[1])
        else:
            n = _t_level(n, P, Q1, W_bu[1, lvl])
    top1 = n

    g1 = jnp.reshape(_sc_gather_pair()(flat(t1), srcR, dstR), (E2, 2 * H))
    edge_agg = _t_edge_final(e1, g1, W_eu[1], ebR)
    naf = _t_segnode(top1, nbR)

    out = _t_final(naf, edge_agg, W1, W2, b1p, W1p, bop)
    return out[:, :1]
